# baseline (speedup 1.0000x reference)
import functools

import jax
import jax.numpy as jnp
from jax import lax
from jax.experimental import pallas as pl
from jax.experimental.pallas import tpu as pltpu, tpu_sc as plsc

N_E = 1024
E_DIM = 64
MU = 0.25
BM = 1024

_SC_INFO = plsc.get_sparse_core_info()
_NC, _NS = _SC_INFO.num_cores, _SC_INFO.num_subcores
_NW = _NC * _NS
_CH = 1152


def _vq_block(x_ref, emb_ref, z2_ref, e2_ref, idx_ref, loss_ref):
    z = x_ref[...]
    emb = emb_ref[...]
    dots = jnp.dot(z, emb.T, preferred_element_type=jnp.float32)
    d = (z2_ref[...] + e2_ref[...]) - 2.0 * dots
    dmin = jnp.min(d, axis=1, keepdims=True)
    colf = jax.lax.broadcasted_iota(jnp.int32, (1, N_E), 1).astype(jnp.float32)
    idxf = jnp.min(jnp.where(d == dmin, colf, float(N_E)), axis=1, keepdims=True)
    idx_ref[...] = idxf.astype(jnp.int32)

    @pl.when(pl.program_id(0) == 0)
    def _():
        loss_ref[...] = jnp.zeros((1, 1), jnp.float32)

    loss_ref[...] += jnp.sum(dmin).reshape(1, 1)


@jax.jit
def _vq_assign(latent, emb, z2, e2):
    m = latent.shape[0]
    return pl.pallas_call(
        _vq_block,
        grid=(m // BM,),
        in_specs=[
            pl.BlockSpec((BM, E_DIM), lambda i: (i, 0)),
            pl.BlockSpec((N_E, E_DIM), lambda i: (0, 0)),
            pl.BlockSpec((BM, 1), lambda i: (i, 0)),
            pl.BlockSpec((1, N_E), lambda i: (0, 0)),
        ],
        out_specs=[
            pl.BlockSpec((BM, 1), lambda i: (i, 0)),
            pl.BlockSpec((1, 1), lambda i: (0, 0)),
        ],
        out_shape=[
            jax.ShapeDtypeStruct((m, 1), jnp.int32),
            jax.ShapeDtypeStruct((1, 1), jnp.float32),
        ],
    )(latent, emb, z2, e2)


def _make_gather(b_total):
    b_per_w = b_total // _NW
    n_chunks = b_per_w // _CH
    mesh = plsc.VectorSubcoreMesh(core_axis_name="c", subcore_axis_name="s")

    @functools.partial(
        pl.kernel, mesh=mesh,
        out_type=jax.ShapeDtypeStruct((b_total, E_DIM), jnp.float32),
        scratch_types=[
            pltpu.VMEM((_CH,), jnp.int32),
            pltpu.VMEM((_CH, E_DIM), jnp.float32),
            pltpu.SemaphoreType.DMA,
        ],
        compiler_params=pltpu.CompilerParams(use_tc_tiling_on_sc=False),
    )
    def gather(table_hbm, idx_hbm, out_hbm, idx_v, rows_v, sem):
        wid = lax.axis_index("s") * _NC + lax.axis_index("c")
        base = wid * b_per_w
        for c in range(n_chunks):
            off = base + c * _CH
            pltpu.sync_copy(idx_hbm.at[pl.ds(off, _CH)], idx_v)
            pltpu.async_copy(table_hbm.at[idx_v], rows_v, sem).wait()
            pltpu.sync_copy(rows_v, out_hbm.at[pl.ds(off, _CH)])

    return gather


_P = 4
_M_TOTAL = 256 * 576
_MP = _M_TOTAL // _P
_gather_kernel = _make_gather(_MP)


def kernel(x, label, idx, emb):
    latent = x.reshape(-1, E_DIM)
    z2 = jnp.sum(latent ** 2, axis=1, keepdims=True)
    e2 = jnp.sum(emb ** 2, axis=1)[None, :]
    xq_parts, idx_parts, loss_sums = [], [], []
    for p in range(_P):
        lat_p = lax.slice_in_dim(latent, p * _MP, (p + 1) * _MP)
        z2_p = lax.slice_in_dim(z2, p * _MP, (p + 1) * _MP)
        idxs2d, loss_sum = _vq_assign(lat_p, emb, z2_p, e2)
        idxs_p = idxs2d.reshape(-1)
        xq_parts.append(_gather_kernel(emb, idxs_p))
        idx_parts.append(idxs_p)
        loss_sums.append(loss_sum[0, 0])
    xq = jnp.concatenate(xq_parts, axis=0)
    idxs = jnp.concatenate(idx_parts, axis=0)
    n = _M_TOTAL * E_DIM
    loss = sum(loss_sums) * ((1.0 + MU) / n)
    x_q_st = xq.reshape(x.shape)
    indices_out = idxs.reshape(x.shape[:-1])
    return (x_q_st, loss, indices_out)


# speedup vs baseline: 1.2804x; 1.2804x over previous
import functools

import jax
import jax.numpy as jnp
from jax import lax
from jax.experimental import pallas as pl
from jax.experimental.pallas import tpu as pltpu, tpu_sc as plsc

N_E = 1024
E_DIM = 64
MU = 0.25
BM = 1024

_SC_INFO = plsc.get_sparse_core_info()
_NC, _NS = _SC_INFO.num_cores, _SC_INFO.num_subcores
_NW = _NC * _NS
_CH = 256


def _vq_block(x_ref, emb_ref, z2_ref, e2_ref, idx_ref, loss_ref):
    z = x_ref[...]
    emb = emb_ref[...]
    dots = jnp.dot(z, emb.T, preferred_element_type=jnp.float32)
    d = (z2_ref[...] + e2_ref[...]) - 2.0 * dots
    dmin = jnp.min(d, axis=1, keepdims=True)
    colf = jax.lax.broadcasted_iota(jnp.int32, (1, N_E), 1).astype(jnp.float32)
    idxf = jnp.min(jnp.where(d == dmin, colf, float(N_E)), axis=1, keepdims=True)
    idx_ref[...] = idxf.astype(jnp.int32)

    @pl.when(pl.program_id(0) == 0)
    def _():
        loss_ref[...] = jnp.zeros((1, 1), jnp.float32)

    loss_ref[...] += jnp.sum(dmin).reshape(1, 1)


@jax.jit
def _vq_assign(latent, emb, z2, e2):
    m = latent.shape[0]
    return pl.pallas_call(
        _vq_block,
        grid=(m // BM,),
        in_specs=[
            pl.BlockSpec((BM, E_DIM), lambda i: (i, 0)),
            pl.BlockSpec((N_E, E_DIM), lambda i: (0, 0)),
            pl.BlockSpec((BM, 1), lambda i: (i, 0)),
            pl.BlockSpec((1, N_E), lambda i: (0, 0)),
        ],
        out_specs=[
            pl.BlockSpec((BM, 1), lambda i: (i, 0)),
            pl.BlockSpec((1, 1), lambda i: (0, 0)),
        ],
        out_shape=[
            jax.ShapeDtypeStruct((m, 1), jnp.int32),
            jax.ShapeDtypeStruct((1, 1), jnp.float32),
        ],
    )(latent, emb, z2, e2)


def _make_gather(b_total):
    b_per_w = b_total // _NW
    n_chunks = b_per_w // _CH
    mesh = plsc.VectorSubcoreMesh(core_axis_name="c", subcore_axis_name="s")

    @functools.partial(
        pl.kernel, mesh=mesh,
        compiler_params=pltpu.CompilerParams(use_tc_tiling_on_sc=False),
        out_type=jax.ShapeDtypeStruct((b_total, E_DIM), jnp.float32),
        scratch_types=[
            pltpu.VMEM((b_per_w,), jnp.int32),
            pltpu.VMEM((_CH, E_DIM), jnp.float32),
            pltpu.VMEM((_CH, E_DIM), jnp.float32),
            pltpu.SemaphoreType.DMA,
            pltpu.SemaphoreType.DMA,
        ],
    )
    def gather(table_hbm, idx_hbm, out_hbm, idx_v, r0, r1, sg, so):
        rb = (r0, r1)
        wid = lax.axis_index("s") * _NC + lax.axis_index("c")
        base = wid * b_per_w
        pltpu.sync_copy(idx_hbm.at[pl.ds(base, b_per_w)], idx_v)
        hg = pltpu.async_copy(table_hbm.at[idx_v.at[pl.ds(0, _CH)]], rb[0], sg)
        hw = None
        for c in range(n_chunks):
            cur = c % 2
            hg.wait()
            if hw is not None:
                hw.wait()
            if c + 1 < n_chunks:
                hg = pltpu.async_copy(
                    table_hbm.at[idx_v.at[pl.ds((c + 1) * _CH, _CH)]],
                    rb[1 - cur], sg)
            hw = pltpu.async_copy(rb[cur],
                                  out_hbm.at[pl.ds(base + c * _CH, _CH)], so)
        hw.wait()

    return gather


_M_TOTAL = 256 * 576
_gather_kernel = _make_gather(_M_TOTAL)


def kernel(x, label, idx, emb):
    latent = x.reshape(-1, E_DIM)
    z2 = jnp.sum(latent ** 2, axis=1, keepdims=True)
    e2 = jnp.sum(emb ** 2, axis=1)[None, :]
    idxs2d, loss_sum = _vq_assign(latent, emb, z2, e2)
    idxs = idxs2d.reshape(-1)
    xq = _gather_kernel(emb, idxs)
    n = _M_TOTAL * E_DIM
    loss = loss_sum[0, 0] * ((1.0 + MU) / n)
    x_q_st = xq.reshape(x.shape)
    indices_out = idxs.reshape(x.shape[:-1])
    return (x_q_st, loss, indices_out)


# speedup vs baseline: 1.5992x; 1.2490x over previous
import functools

import jax
import jax.numpy as jnp
from jax import lax
from jax.experimental import pallas as pl
from jax.experimental.pallas import tpu as pltpu, tpu_sc as plsc

N_E = 1024
E_DIM = 64
MU = 0.25
BM = 1024

_SC_INFO = plsc.get_sparse_core_info()
_NC, _NS = _SC_INFO.num_cores, _SC_INFO.num_subcores
_NW = _NC * _NS
_CH = 256


def _vq_block(x_ref, emb_ref, z2_ref, e2_ref, idx_ref, loss_ref):
    z = x_ref[...]
    emb = emb_ref[...]
    dots = lax.dot_general(emb, z, (((1,), (1,)), ((), ())),
                           preferred_element_type=jnp.float32)
    d = (z2_ref[...] + e2_ref[...]) - 2.0 * dots
    dmin = jnp.min(d, axis=0, keepdims=True)
    rowf = jax.lax.broadcasted_iota(jnp.int32, (N_E, 1), 0).astype(jnp.float32)
    idxf = jnp.min(jnp.where(d == dmin, rowf, float(N_E)), axis=0, keepdims=True)
    idx_ref[...] = idxf.astype(jnp.int32)[None]

    @pl.when(pl.program_id(0) == 0)
    def _():
        loss_ref[...] = jnp.zeros((1, 1), jnp.float32)

    loss_ref[...] += jnp.sum(dmin).reshape(1, 1)


@jax.jit
def _vq_assign(latent, emb, z2, e2):
    m = latent.shape[0]
    return pl.pallas_call(
        _vq_block,
        grid=(m // BM,),
        in_specs=[
            pl.BlockSpec((BM, E_DIM), lambda i: (i, 0)),
            pl.BlockSpec((N_E, E_DIM), lambda i: (0, 0)),
            pl.BlockSpec((1, BM), lambda i: (0, i)),
            pl.BlockSpec((N_E, 1), lambda i: (0, 0)),
        ],
        out_specs=[
            pl.BlockSpec((1, 1, BM), lambda i: (i, 0, 0)),
            pl.BlockSpec((1, 1), lambda i: (0, 0)),
        ],
        out_shape=[
            jax.ShapeDtypeStruct((m // BM, 1, BM), jnp.int32),
            jax.ShapeDtypeStruct((1, 1), jnp.float32),
        ],
    )(latent, emb, z2, e2)


def _make_gather(b_total):
    b_per_w = b_total // _NW
    n_chunks = b_per_w // _CH
    mesh = plsc.VectorSubcoreMesh(core_axis_name="c", subcore_axis_name="s")

    @functools.partial(
        pl.kernel, mesh=mesh,
        out_type=jax.ShapeDtypeStruct((b_total, 128), jnp.float32),
        scratch_types=[
            pltpu.VMEM((b_per_w,), jnp.int32),
            pltpu.VMEM((_CH, 128), jnp.float32),
            pltpu.VMEM((_CH, 128), jnp.float32),
            pltpu.SemaphoreType.DMA,
            pltpu.SemaphoreType.DMA,
        ],
    )
    def gather(table_hbm, idx_hbm, out_hbm, idx_v, r0, r1, sg, so):
        rb = (r0, r1)
        wid = lax.axis_index("s") * _NC + lax.axis_index("c")
        base = wid * b_per_w
        pltpu.sync_copy(idx_hbm.at[pl.ds(base, b_per_w)], idx_v)
        hg = pltpu.async_copy(table_hbm.at[idx_v.at[pl.ds(0, _CH)]], rb[0], sg)
        hw = None
        for c in range(n_chunks):
            cur = c % 2
            hg.wait()
            if hw is not None:
                hw.wait()
            if c + 1 < n_chunks:
                hg = pltpu.async_copy(
                    table_hbm.at[idx_v.at[pl.ds((c + 1) * _CH, _CH)]],
                    rb[1 - cur], sg)
            hw = pltpu.async_copy(rb[cur],
                                  out_hbm.at[pl.ds(base + c * _CH, _CH)], so)
        hw.wait()

    return gather


_M_TOTAL = 256 * 576
_gather_kernel = _make_gather(_M_TOTAL)


def kernel(x, label, idx, emb):
    latent = x.reshape(-1, E_DIM)
    z2 = jnp.sum(latent ** 2, axis=1)[None, :]
    e2 = jnp.sum(emb ** 2, axis=1, keepdims=True)
    idxs2d, loss_sum = _vq_assign(latent, emb, z2, e2)
    idxs = idxs2d.reshape(-1)
    emb128 = jnp.pad(emb, ((0, 0), (0, 128 - E_DIM)))
    xq = _gather_kernel(emb128, idxs)[:, :E_DIM]
    n = _M_TOTAL * E_DIM
    loss = loss_sum[0, 0] * ((1.0 + MU) / n)
    x_q_st = xq.reshape(x.shape)
    indices_out = idxs.reshape(x.shape[:-1])
    return (x_q_st, loss, indices_out)


# speedup vs baseline: 1.6053x; 1.0038x over previous
import functools

import jax
import jax.numpy as jnp
from jax import lax
from jax.experimental import pallas as pl
from jax.experimental.pallas import tpu as pltpu, tpu_sc as plsc

N_E = 1024
E_DIM = 64
MU = 0.25
BM = 1024

_SC_INFO = plsc.get_sparse_core_info()
_NC, _NS = _SC_INFO.num_cores, _SC_INFO.num_subcores
_NW = _NC * _NS
_CH = 256


def _vq_block(x_ref, emb_ref, z2_ref, e2_ref, idx_ref, loss_ref):
    z = x_ref[...]
    emb = emb_ref[...]
    dots = lax.dot_general(emb, z, (((1,), (1,)), ((), ())),
                           preferred_element_type=jnp.float32)
    d = (z2_ref[...] + e2_ref[...]) - 2.0 * dots
    dmin = jnp.min(d, axis=0, keepdims=True)
    rowf = jax.lax.broadcasted_iota(jnp.int32, (N_E, 1), 0).astype(jnp.float32)
    idxf = jnp.min(jnp.where(d == dmin, rowf, float(N_E)), axis=0, keepdims=True)
    idx_ref[...] = idxf.astype(jnp.int32).reshape(BM // 128, 128)

    @pl.when(pl.program_id(0) == 0)
    def _():
        loss_ref[...] = jnp.zeros((1, 1), jnp.float32)

    loss_ref[...] += jnp.sum(dmin).reshape(1, 1)


@jax.jit
def _vq_assign(latent, emb, z2, e2):
    m = latent.shape[0]
    return pl.pallas_call(
        _vq_block,
        grid=(m // BM,),
        in_specs=[
            pl.BlockSpec((BM, E_DIM), lambda i: (i, 0)),
            pl.BlockSpec((N_E, E_DIM), lambda i: (0, 0)),
            pl.BlockSpec((1, BM), lambda i: (0, i)),
            pl.BlockSpec((N_E, 1), lambda i: (0, 0)),
        ],
        out_specs=[
            pl.BlockSpec((BM // 128, 128), lambda i: (i, 0)),
            pl.BlockSpec((1, 1), lambda i: (0, 0)),
        ],
        out_shape=[
            jax.ShapeDtypeStruct((m // 128, 128), jnp.int32),
            jax.ShapeDtypeStruct((1, 1), jnp.float32),
        ],
    )(latent, emb, z2, e2)


def _make_gather(b_total):
    b_per_w = b_total // _NW
    n_chunks = b_per_w // _CH
    mesh = plsc.VectorSubcoreMesh(core_axis_name="c", subcore_axis_name="s")

    @functools.partial(
        pl.kernel, mesh=mesh,
        out_type=jax.ShapeDtypeStruct((b_total, 128), jnp.float32),
        scratch_types=[
            pltpu.VMEM((b_per_w,), jnp.int32),
            pltpu.VMEM((_CH, 128), jnp.float32),
            pltpu.VMEM((_CH, 128), jnp.float32),
            pltpu.SemaphoreType.DMA,
            pltpu.SemaphoreType.DMA,
        ],
    )
    def gather(table_hbm, idx_hbm, out_hbm, idx_v, r0, r1, sg, so):
        rb = (r0, r1)
        wid = lax.axis_index("s") * _NC + lax.axis_index("c")
        base = wid * b_per_w
        pltpu.sync_copy(idx_hbm.at[pl.ds(base, b_per_w)], idx_v)
        hg = pltpu.async_copy(table_hbm.at[idx_v.at[pl.ds(0, _CH)]], rb[0], sg)
        hw = None
        for c in range(n_chunks):
            cur = c % 2
            hg.wait()
            if hw is not None:
                hw.wait()
            if c + 1 < n_chunks:
                hg = pltpu.async_copy(
                    table_hbm.at[idx_v.at[pl.ds((c + 1) * _CH, _CH)]],
                    rb[1 - cur], sg)
            hw = pltpu.async_copy(rb[cur],
                                  out_hbm.at[pl.ds(base + c * _CH, _CH)], so)
        hw.wait()

    return gather


_M_TOTAL = 256 * 576
_gather_kernel = _make_gather(_M_TOTAL)


def kernel(x, label, idx, emb):
    latent = x.reshape(-1, E_DIM)
    z2 = jnp.sum(latent ** 2, axis=1)[None, :]
    e2 = jnp.sum(emb ** 2, axis=1, keepdims=True)
    idxs2d, loss_sum = _vq_assign(latent, emb, z2, e2)
    idxs = idxs2d.reshape(-1)
    emb128 = jnp.pad(emb, ((0, 0), (0, 128 - E_DIM)))
    xq = _gather_kernel(emb128, idxs)[:, :E_DIM]
    n = _M_TOTAL * E_DIM
    loss = loss_sum[0, 0] * ((1.0 + MU) / n)
    x_q_st = xq.reshape(x.shape)
    indices_out = idxs.reshape(x.shape[:-1])
    return (x_q_st, loss, indices_out)


# speedup vs baseline: 1.6617x; 1.0351x over previous
import functools

import jax
import jax.numpy as jnp
from jax import lax
from jax.experimental import pallas as pl
from jax.experimental.pallas import tpu as pltpu, tpu_sc as plsc

N_E = 1024
E_DIM = 64
MU = 0.25
BM = 1024

_SC_INFO = plsc.get_sparse_core_info()
_NC, _NS = _SC_INFO.num_cores, _SC_INFO.num_subcores
_NW = _NC * _NS
_CH = 256


def _vq_block(x_ref, emb_ref, z2_ref, e2_ref, idx_ref, loss_ref):
    z = x_ref[...]
    emb = emb_ref[...]
    dots = lax.dot_general(emb, z, (((1,), (1,)), ((), ())),
                           preferred_element_type=jnp.float32)
    d = (z2_ref[...] + e2_ref[...]) - 2.0 * dots
    dmin = jnp.min(d, axis=0, keepdims=True)
    rowf = jax.lax.broadcasted_iota(jnp.int32, (N_E, 1), 0).astype(jnp.float32)
    idxf = jnp.min(jnp.where(d == dmin, rowf, float(N_E)), axis=0, keepdims=True)
    idx_ref[...] = idxf.astype(jnp.int32).reshape(BM // 128, 128)
    loss_ref[...] = jnp.sum(dmin).reshape(1, 1, 1)


@jax.jit
def _vq_assign(latent, emb, z2, e2):
    m = latent.shape[0]
    return pl.pallas_call(
        _vq_block,
        grid=(m // BM,),
        compiler_params=pltpu.CompilerParams(
            dimension_semantics=("parallel",)),
        in_specs=[
            pl.BlockSpec((BM, E_DIM), lambda i: (i, 0)),
            pl.BlockSpec((N_E, E_DIM), lambda i: (0, 0)),
            pl.BlockSpec((1, BM), lambda i: (0, i)),
            pl.BlockSpec((N_E, 1), lambda i: (0, 0)),
        ],
        out_specs=[
            pl.BlockSpec((BM // 128, 128), lambda i: (i, 0)),
            pl.BlockSpec((1, 1, 1), lambda i: (i, 0, 0)),
        ],
        out_shape=[
            jax.ShapeDtypeStruct((m // 128, 128), jnp.int32),
            jax.ShapeDtypeStruct((m // BM, 1, 1), jnp.float32),
        ],
    )(latent, emb, z2, e2)


def _make_gather(b_total):
    b_per_w = b_total // _NW
    n_chunks = b_per_w // _CH
    mesh = plsc.VectorSubcoreMesh(core_axis_name="c", subcore_axis_name="s")

    @functools.partial(
        pl.kernel, mesh=mesh,
        out_type=jax.ShapeDtypeStruct((b_total, 128), jnp.float32),
        scratch_types=[
            pltpu.VMEM((b_per_w,), jnp.int32),
            pltpu.VMEM((_CH, 128), jnp.float32),
            pltpu.VMEM((_CH, 128), jnp.float32),
            pltpu.SemaphoreType.DMA,
            pltpu.SemaphoreType.DMA,
        ],
    )
    def gather(table_hbm, idx_hbm, out_hbm, idx_v, r0, r1, sg, so):
        rb = (r0, r1)
        wid = lax.axis_index("s") * _NC + lax.axis_index("c")
        base = wid * b_per_w
        pltpu.sync_copy(idx_hbm.at[pl.ds(base, b_per_w)], idx_v)
        hg = pltpu.async_copy(table_hbm.at[idx_v.at[pl.ds(0, _CH)]], rb[0], sg)
        hw = None
        for c in range(n_chunks):
            cur = c % 2
            hg.wait()
            if hw is not None:
                hw.wait()
            if c + 1 < n_chunks:
                hg = pltpu.async_copy(
                    table_hbm.at[idx_v.at[pl.ds((c + 1) * _CH, _CH)]],
                    rb[1 - cur], sg)
            hw = pltpu.async_copy(rb[cur],
                                  out_hbm.at[pl.ds(base + c * _CH, _CH)], so)
        hw.wait()

    return gather


_M_TOTAL = 256 * 576
_gather_kernel = _make_gather(_M_TOTAL)


def kernel(x, label, idx, emb):
    latent = x.reshape(-1, E_DIM)
    z2 = jnp.sum(latent ** 2, axis=1)[None, :]
    e2 = jnp.sum(emb ** 2, axis=1, keepdims=True)
    idxs2d, loss_sum = _vq_assign(latent, emb, z2, e2)
    idxs = idxs2d.reshape(-1)
    emb128 = jnp.pad(emb, ((0, 0), (0, 128 - E_DIM)))
    xq = _gather_kernel(emb128, idxs)[:, :E_DIM]
    n = _M_TOTAL * E_DIM
    loss = jnp.sum(loss_sum) * ((1.0 + MU) / n)
    x_q_st = xq.reshape(x.shape)
    indices_out = idxs.reshape(x.shape[:-1])
    return (x_q_st, loss, indices_out)


# speedup vs baseline: 1.6641x; 1.0015x over previous
import functools

import jax
import jax.numpy as jnp
from jax import lax
from jax.experimental import pallas as pl
from jax.experimental.pallas import tpu as pltpu, tpu_sc as plsc

N_E = 1024
E_DIM = 64
MU = 0.25
BM = 1024

_SC_INFO = plsc.get_sparse_core_info()
_NC, _NS = _SC_INFO.num_cores, _SC_INFO.num_subcores
_NW = _NC * _NS
_CH = 256


def _vq_block(x_ref, emb_ref, z2_ref, e2_ref, idx_ref, loss_ref):
    z = x_ref[...]
    emb = emb_ref[...]
    dots = lax.dot_general(emb, z, (((1,), (1,)), ((), ())),
                           preferred_element_type=jnp.float32)
    d = (z2_ref[...] + e2_ref[...]) - 2.0 * dots
    dmin = jnp.min(d, axis=0, keepdims=True)
    rowf = jax.lax.broadcasted_iota(jnp.int32, (N_E, 1), 0).astype(jnp.float32)
    idxf = jnp.min(jnp.where(d == dmin, rowf, float(N_E)), axis=0, keepdims=True)
    idx_ref[...] = idxf.astype(jnp.int32).reshape(BM // 128, 128)
    loss_ref[...] = jnp.sum(dmin).reshape(1, 1, 1)


def _vq_assign(latent, emb, z2, e2, off, nblk):
    return pl.pallas_call(
        _vq_block,
        grid=(nblk,),
        compiler_params=pltpu.CompilerParams(
            dimension_semantics=("parallel",)),
        in_specs=[
            pl.BlockSpec((BM, E_DIM), lambda i: (i + off, 0)),
            pl.BlockSpec((N_E, E_DIM), lambda i: (0, 0)),
            pl.BlockSpec((1, BM), lambda i: (0, i + off)),
            pl.BlockSpec((N_E, 1), lambda i: (0, 0)),
        ],
        out_specs=[
            pl.BlockSpec((BM // 128, 128), lambda i: (i, 0)),
            pl.BlockSpec((1, 1, 1), lambda i: (i, 0, 0)),
        ],
        out_shape=[
            jax.ShapeDtypeStruct((nblk * BM // 128, 128), jnp.int32),
            jax.ShapeDtypeStruct((nblk, 1, 1), jnp.float32),
        ],
    )(latent, emb, z2, e2)


def _make_gather(b_total):
    b_per_w = b_total // _NW
    n_chunks = b_per_w // _CH
    mesh = plsc.VectorSubcoreMesh(core_axis_name="c", subcore_axis_name="s")

    @functools.partial(
        pl.kernel, mesh=mesh,
        out_type=jax.ShapeDtypeStruct((b_total, 128), jnp.float32),
        scratch_types=[
            pltpu.VMEM((b_per_w,), jnp.int32),
            pltpu.VMEM((_CH, 128), jnp.float32),
            pltpu.VMEM((_CH, 128), jnp.float32),
            pltpu.SemaphoreType.DMA,
            pltpu.SemaphoreType.DMA,
        ],
    )
    def gather(table_hbm, idx_hbm, out_hbm, idx_v, r0, r1, sg, so):
        rb = (r0, r1)
        wid = lax.axis_index("s") * _NC + lax.axis_index("c")
        base = wid * b_per_w
        pltpu.sync_copy(idx_hbm.at[pl.ds(base, b_per_w)], idx_v)
        hg = pltpu.async_copy(table_hbm.at[idx_v.at[pl.ds(0, _CH)]], rb[0], sg)
        hw = None
        for c in range(n_chunks):
            cur = c % 2
            hg.wait()
            if hw is not None:
                hw.wait()
            if c + 1 < n_chunks:
                hg = pltpu.async_copy(
                    table_hbm.at[idx_v.at[pl.ds((c + 1) * _CH, _CH)]],
                    rb[1 - cur], sg)
            hw = pltpu.async_copy(rb[cur],
                                  out_hbm.at[pl.ds(base + c * _CH, _CH)], so)
        hw.wait()

    return gather


_M_TOTAL = 256 * 576
_NBLK = _M_TOTAL // BM
_gather_kernel = _make_gather(_M_TOTAL)


def kernel(x, label, idx, emb):
    latent = x.reshape(-1, E_DIM)
    z2 = jnp.sum(latent ** 2, axis=1)[None, :]
    e2 = jnp.sum(emb ** 2, axis=1, keepdims=True)
    emb128 = jnp.pad(emb, ((0, 0), (0, 128 - E_DIM)))
    idxs2d, loss_blocks = _vq_assign(latent, emb, z2, e2, 0, _NBLK)
    idxs = idxs2d.reshape(-1)
    xq = _gather_kernel(emb128, idxs)[:, :E_DIM]
    n = _M_TOTAL * E_DIM
    loss = jnp.sum(loss_blocks) * ((1.0 + MU) / n)
    x_q_st = xq.reshape(x.shape)
    indices_out = idxs.reshape(x.shape[:-1])
    return (x_q_st, loss, indices_out)
